# Initial kernel scaffold; baseline (speedup 1.0000x reference)
import jax
import jax.numpy as jnp
from jax.experimental import pallas as pl


def kernel(fmess, bgraph, Wz_w, Wz_b, Wr_w, Ur_w, Ur_b, Wh_w, Wh_b):
    raise NotImplementedError("write your pallas kernel here")


# speedup vs baseline: 22.8095x; 22.8095x over previous
import functools

import jax
import jax.numpy as jnp
from jax import lax
from jax.experimental import pallas as pl
from jax.experimental.pallas import tpu as pltpu
from jax.experimental.pallas import tpu_sc as plsc


def _precompute_body(fm_ref, wz_ref, wr_ref, wh_ref, bz_ref, bh_ref,
                     fz_ref, fr_ref, fh_ref, h1_ref, *, block_rows):
    fm = fm_ref[...]
    fz = jnp.dot(fm, wz_ref[...], preferred_element_type=jnp.float32)
    fr = jnp.dot(fm, wr_ref[...], preferred_element_type=jnp.float32)
    fh = jnp.dot(fm, wh_ref[...], preferred_element_type=jnp.float32)
    fz_ref[...] = fz
    fr_ref[...] = fr
    fh_ref[...] = fh
    z1 = jax.nn.sigmoid(fz + bz_ref[...])
    p1 = jnp.tanh(fh + bh_ref[...])
    h1 = z1 * p1
    row = lax.broadcasted_iota(jnp.int32, h1.shape, 0) + pl.program_id(0) * block_rows
    h1_ref[...] = jnp.where(row == 0, 0.0, h1)


def _dense_body(hn_ref, fz_ref, fr_ref, fh_ref, uz_ref, ur_ref, uh_ref,
                bz_ref, urb_ref, bh_ref, out_ref, *, n_nei, block_rows):
    fr = fr_ref[...]
    ur = ur_ref[...]
    urb = urb_ref[...]
    sum_h = jnp.zeros_like(fr)
    sum_g = jnp.zeros_like(fr)
    for n in range(n_nei):
        hn = hn_ref[n]
        r = jax.nn.sigmoid(
            fr + jnp.dot(hn, ur, preferred_element_type=jnp.float32) + urb)
        sum_h = sum_h + hn
        sum_g = sum_g + r * hn
    z = jax.nn.sigmoid(
        fz_ref[...] + jnp.dot(sum_h, uz_ref[...], preferred_element_type=jnp.float32)
        + bz_ref[...])
    pre = jnp.tanh(
        fh_ref[...] + jnp.dot(sum_g, uh_ref[...], preferred_element_type=jnp.float32)
        + bh_ref[...])
    nh = (1.0 - z) * sum_h + z * pre
    row = lax.broadcasted_iota(jnp.int32, nh.shape, 0) + pl.program_id(0) * block_rows
    out_ref[...] = jnp.where(row == 0, 0.0, nh)


def _make_gather(e_total, h_size, n_nei):
    info = plsc.get_sparse_core_info()
    nc, ns = info.num_cores, info.num_subcores
    nw = nc * ns
    total = e_total * n_nei
    per_w = total // nw
    assert per_w * nw == total and per_w % 8 == 0
    k = 2000
    while per_w % k != 0:
        k //= 2
    n_chunks = per_w // k

    mesh = plsc.VectorSubcoreMesh(core_axis_name="c", subcore_axis_name="s")

    @functools.partial(
        pl.kernel,
        mesh=mesh,
        out_type=jax.ShapeDtypeStruct((total, h_size), jnp.float32),
        scratch_types=[
            pltpu.VMEM((k,), jnp.int32),
            pltpu.VMEM((k, h_size), jnp.float32),
            pltpu.SemaphoreType.DMA,
        ],
        compiler_params=pltpu.CompilerParams(use_tc_tiling_on_sc=False),
    )
    def gather(h_hbm, idx_hbm, out_hbm, idx_v, rows_v, sem):
        wid = lax.axis_index("s") * nc + lax.axis_index("c")
        base = wid * per_w

        def body(i, carry):
            off = base + i * k
            pltpu.sync_copy(idx_hbm.at[pl.ds(off, k)], idx_v)
            pltpu.async_copy(h_hbm.at[idx_v], rows_v, sem).wait()
            pltpu.sync_copy(rows_v, out_hbm.at[pl.ds(off, k)])
            return carry

        lax.fori_loop(0, n_chunks, body, 0)

    return gather


def kernel(fmess, bgraph, Wz_w, Wz_b, Wr_w, Ur_w, Ur_b, Wh_w, Wh_b):
    e_total, in_size = fmess.shape
    h_size = Ur_w.shape[0]
    n_nei = bgraph.shape[1]
    depth = 3

    wzf_t = Wz_w[:, :in_size].T
    wzh_t = Wz_w[:, in_size:].T
    whf_t = Wh_w[:, :in_size].T
    whh_t = Wh_w[:, in_size:].T
    wr_t = Wr_w.T
    ur_t = Ur_w.T
    bz = Wz_b.reshape(1, h_size)
    urb = Ur_b.reshape(1, h_size)
    bh = Wh_b.reshape(1, h_size)
    idx = bgraph.T.reshape(-1)

    bp = 2000
    grid_p = e_total // bp
    fz, fr, fh, h = pl.pallas_call(
        functools.partial(_precompute_body, block_rows=bp),
        grid=(grid_p,),
        in_specs=[
            pl.BlockSpec((bp, in_size), lambda i: (i, 0)),
            pl.BlockSpec((in_size, h_size), lambda i: (0, 0)),
            pl.BlockSpec((in_size, h_size), lambda i: (0, 0)),
            pl.BlockSpec((in_size, h_size), lambda i: (0, 0)),
            pl.BlockSpec((1, h_size), lambda i: (0, 0)),
            pl.BlockSpec((1, h_size), lambda i: (0, 0)),
        ],
        out_specs=[pl.BlockSpec((bp, h_size), lambda i: (i, 0))] * 4,
        out_shape=[jax.ShapeDtypeStruct((e_total, h_size), jnp.float32)] * 4,
        compiler_params=pltpu.CompilerParams(
            dimension_semantics=("parallel",)),
    )(fmess, wzf_t, wr_t, whf_t, bz, bh)

    gather = _make_gather(e_total, h_size, n_nei)

    bd = 2000
    grid_d = e_total // bd
    dense = pl.pallas_call(
        functools.partial(_dense_body, n_nei=n_nei, block_rows=bd),
        grid=(grid_d,),
        in_specs=[
            pl.BlockSpec((n_nei, bd, h_size), lambda i: (0, i, 0)),
            pl.BlockSpec((bd, h_size), lambda i: (i, 0)),
            pl.BlockSpec((bd, h_size), lambda i: (i, 0)),
            pl.BlockSpec((bd, h_size), lambda i: (i, 0)),
            pl.BlockSpec((h_size, h_size), lambda i: (0, 0)),
            pl.BlockSpec((h_size, h_size), lambda i: (0, 0)),
            pl.BlockSpec((h_size, h_size), lambda i: (0, 0)),
            pl.BlockSpec((1, h_size), lambda i: (0, 0)),
            pl.BlockSpec((1, h_size), lambda i: (0, 0)),
            pl.BlockSpec((1, h_size), lambda i: (0, 0)),
        ],
        out_specs=pl.BlockSpec((bd, h_size), lambda i: (i, 0)),
        out_shape=jax.ShapeDtypeStruct((e_total, h_size), jnp.float32),
        compiler_params=pltpu.CompilerParams(
            dimension_semantics=("parallel",)),
    )

    for _ in range(depth - 1):
        hnei = gather(h, idx)
        hnei = hnei.reshape(n_nei, e_total, h_size)
        h = dense(hnei, fz, fr, fh, wzh_t, ur_t, whh_t, bz, urb, bh)

    return h
